# baseline (speedup 1.0000x reference)
import jax
import jax.numpy as jnp
from jax.experimental import pallas as pl
from jax.experimental.pallas import tpu as pltpu

_PA = 64
_PB = 72
_ROWS = 3280


def _router_kernel(x_ref, w1_ref, b1_ref, w2_ref, b2_ref,
                   probs_ref, idx_ref, val_ref):
    m = jnp.mean(x_ref[...], axis=1)
    z = jnp.maximum(
        jnp.dot(m, w1_ref[...], preferred_element_type=jnp.float32)
        + b1_ref[...], 0.0)
    logits = jnp.dot(z, w2_ref[...],
                     preferred_element_type=jnp.float32) + b2_ref[...]
    probs = jax.nn.softmax(logits, axis=1)
    E = probs.shape[1]
    col = jax.lax.broadcasted_iota(jnp.int32, probs.shape, 1)
    v1 = jnp.max(probs, axis=1, keepdims=True)
    i1 = jnp.min(jnp.where(probs == v1, col, E), axis=1, keepdims=True)
    masked = jnp.where(col == i1, -jnp.inf, probs)
    v2 = jnp.max(masked, axis=1, keepdims=True)
    i2 = jnp.min(jnp.where(masked == v2, col, E), axis=1, keepdims=True)
    probs_ref[...] = probs
    idx_ref[...] = jnp.concatenate([i1, i2], axis=1)
    val_ref[...] = jnp.concatenate([v1, v2], axis=1)


def _moe_kernel(idx_ref, val_ref, x_ref, w_ref, beta_ref, out_ref,
                ca_ref, cb_ref, la_ref, lb_ref, ra_ref, rb_ref):
    b = pl.program_id(0)
    k = pl.program_id(1)
    HW, C = x_ref.shape[1], x_ref.shape[2]
    W = 56
    xx = x_ref[0]

    @pl.when(k == 0)
    def _build():
        xb = xx.astype(jnp.bfloat16)
        row = jax.lax.broadcasted_iota(jnp.int32, (HW, 1), 0) % W
        m_l = (row != 0).astype(jnp.bfloat16)
        m_r = (row != W - 1).astype(jnp.bfloat16)
        zrow = jnp.zeros((1, C), jnp.bfloat16)
        sh_l = jnp.concatenate([zrow, xb[:-1, :]], axis=0) * m_l
        sh_r = jnp.concatenate([xb[1:, :], zrow], axis=0) * m_r
        for ref, phase, mid in ((ca_ref, _PA, xb), (cb_ref, _PB, xb),
                                (la_ref, _PA, sh_l), (lb_ref, _PB, sh_l),
                                (ra_ref, _PA, sh_r), (rb_ref, _PB, sh_r)):
            ref[0:phase, :] = jnp.zeros((phase, C), jnp.bfloat16)
            ref[pl.ds(phase, HW), :] = mid
            ref[pl.ds(phase + HW, _ROWS - phase - HW), :] = (
                jnp.zeros((_ROWS - phase - HW, C), jnp.bfloat16))

    rw = val_ref[b * 2 + k]
    acc = jnp.zeros((HW, C), jnp.float32)
    for t in range(9):
        dy, dx = t // 3 - 1, t % 3 - 1
        if dy == 0:
            buf, off = ((la_ref, ca_ref, ra_ref)[dx + 1], _PA)
        else:
            buf, off = ((lb_ref, cb_ref, rb_ref)[dx + 1], _PB + dy * W)
        sh = buf[pl.ds(off, HW), :]
        acc = acc + jnp.dot(sh, w_ref[0, t],
                            preferred_element_type=jnp.float32)
    o = jnp.maximum(acc + beta_ref[0], 0.0) * rw

    @pl.when(k == 0)
    def _init():
        out_ref[0] = xx + o

    @pl.when(k != 0)
    def _accum():
        out_ref[0] = out_ref[0] + o


def kernel(x, fc1_w, fc1_b, fc2_w, fc2_b, conv_w, bn_gamma, bn_beta):
    B, C, H, W = x.shape
    E, HID = fc2_w.shape[0], fc1_w.shape[0]
    HW = H * W
    K = 2
    xt = x.reshape(B, C, HW).transpose(0, 2, 1)

    probs, idx2, val2 = pl.pallas_call(
        _router_kernel,
        out_shape=[
            jax.ShapeDtypeStruct((B, E), jnp.float32),
            jax.ShapeDtypeStruct((B, K), jnp.int32),
            jax.ShapeDtypeStruct((B, K), jnp.float32),
        ],
    )(xt, fc1_w.T, fc1_b.reshape(1, HID), fc2_w.T, fc2_b.reshape(1, E))

    eps = 1e-5
    scale = bn_gamma * (1.0 / jnp.sqrt(1.0 + eps))
    wt = conv_w.transpose(0, 3, 4, 2, 1).reshape(E, 9, C, C)
    wt = (wt * scale[:, None, None, :]).astype(jnp.bfloat16)
    beta3 = bn_beta.reshape(E, 1, C)

    grid_spec = pltpu.PrefetchScalarGridSpec(
        num_scalar_prefetch=2,
        grid=(B, K),
        in_specs=[
            pl.BlockSpec((1, HW, C), lambda b, k, idx, val: (b, 0, 0)),
            pl.BlockSpec((1, 9, C, C),
                         lambda b, k, idx, val: (idx[b * 2 + k], 0, 0, 0)),
            pl.BlockSpec((1, 1, C),
                         lambda b, k, idx, val: (idx[b * 2 + k], 0, 0)),
        ],
        out_specs=pl.BlockSpec((1, HW, C), lambda b, k, idx, val: (b, 0, 0)),
        scratch_shapes=[pltpu.VMEM((_ROWS, C), jnp.bfloat16)] * 6,
    )
    out_t = pl.pallas_call(
        _moe_kernel,
        grid_spec=grid_spec,
        out_shape=jax.ShapeDtypeStruct((B, HW, C), jnp.float32),
    )(idx2.reshape(B * K), val2.reshape(B * K), xt, wt, beta3)

    out = out_t.transpose(0, 2, 1).reshape(B, C, H, W)
    return (out, probs)
